# baseline (speedup 1.0000x reference)
import functools

import jax
import jax.numpy as jnp
from jax import lax
from jax.experimental import pallas as pl
from jax.experimental.pallas import tpu as pltpu
from jax.experimental.pallas import tpu_sc as plsc

EMBEDDING_DIM = 512
NUM_CORES = 2
NUM_SUBCORES = 16
NUM_WORKERS = NUM_CORES * NUM_SUBCORES
CHUNK = 32
NBUF = 6


@functools.lru_cache(maxsize=None)
def _make_gather(batch: int):
    rows_per_worker = batch // NUM_WORKERS
    n_chunks = rows_per_worker // CHUNK
    assert rows_per_worker % CHUNK == 0
    tail = n_chunks % NBUF
    steady = n_chunks - NBUF - tail
    assert steady >= 0 and steady % NBUF == 0

    mesh = plsc.VectorSubcoreMesh(
        core_axis_name="c", subcore_axis_name="s",
        num_cores=NUM_CORES, num_subcores=NUM_SUBCORES)

    @functools.partial(
        pl.kernel,
        mesh=mesh,
        out_type=jax.ShapeDtypeStruct((batch, EMBEDDING_DIM), jnp.float32),
        scratch_types=(
            [pltpu.VMEM((rows_per_worker,), jnp.int32)]
            + [pltpu.VMEM((CHUNK, EMBEDDING_DIM), jnp.float32)] * NBUF
            + [pltpu.SemaphoreType.DMA] * (2 * NBUF)
        ),
    )
    def gather_kernel(table_hbm, idx_hbm, out_hbm, idx_v, *bufs_and_sems):
        rows = bufs_and_sems[:NBUF]
        gsems = bufs_and_sems[NBUF:2 * NBUF]
        ssems = bufs_and_sems[2 * NBUF:]
        wid = lax.axis_index("s") * NUM_CORES + lax.axis_index("c")
        base = wid * rows_per_worker
        pltpu.sync_copy(idx_hbm.at[pl.ds(base, rows_per_worker)], idx_v)

        def start_gather(k, off):
            pltpu.async_copy(
                table_hbm.at[idx_v.at[pl.ds(off, CHUNK)]], rows[k], gsems[k])

        def wait_gather(k, off):
            pltpu.make_async_copy(
                table_hbm.at[idx_v.at[pl.ds(off, CHUNK)]], rows[k],
                gsems[k]).wait()

        def start_scatter(k, off):
            pltpu.async_copy(
                rows[k], out_hbm.at[pl.ds(base + off, CHUNK)], ssems[k])

        def wait_scatter(k, off):
            pltpu.make_async_copy(
                rows[k], out_hbm.at[pl.ds(base + off, CHUNK)], ssems[k]).wait()

        for k in range(NBUF):
            start_gather(k, k * CHUNK)

        @pl.loop(0, steady, step=NBUF)
        def _chunk(g):
            off = g * CHUNK
            for k in range(NBUF):
                wait_gather(k, off + k * CHUNK)
                start_scatter(k, off + k * CHUNK)
            for k in range(NBUF):
                wait_scatter(k, off + k * CHUNK)
                start_gather(k, off + (k + NBUF) * CHUNK)

        off = steady * CHUNK
        for k in range(NBUF):
            wait_gather(k, off + k * CHUNK)
            start_scatter(k, off + k * CHUNK)
        for k in range(NBUF):
            wait_scatter(k, off + k * CHUNK)
            if k < tail:
                start_gather(k, off + (k + NBUF) * CHUNK)
        off += NBUF * CHUNK
        for k in range(tail):
            wait_gather(k, off + k * CHUNK)
            start_scatter(k, off + k * CHUNK)
        for k in range(tail):
            wait_scatter(k, off + k * CHUNK)

    return gather_kernel


def kernel(indices, embeddings):
    batch = indices.size
    idx_flat = indices.reshape(batch).astype(jnp.int32)
    out = _make_gather(batch)(embeddings, idx_flat)
    return out.reshape(*indices.shape, EMBEDDING_DIM)
